# Initial kernel scaffold; baseline (speedup 1.0000x reference)
import jax
import jax.numpy as jnp
from jax.experimental import pallas as pl


def kernel(node_feat, edge_feat, edge_index, W1_0, b1_0, W2_0, b2_0, W1, b1, W2, b2, Wout, bout):
    raise NotImplementedError("write your pallas kernel here")


# speedup vs baseline: 5.3898x; 5.3898x over previous
import functools

import jax
import jax.numpy as jnp
from jax import lax
from jax.experimental import pallas as pl
from jax.experimental.pallas import tpu as pltpu
from jax.experimental.pallas import tpu_sc as plsc

N = 10000
E = 320000
G = 100
NPG = 100
D_NODE = 128
D_EDGE = 16
LATENT = 32
NUM_LAYERS = 5
K = 30
OUT_DIM = 128

NTILES = 32
EPT = E // NTILES
CHUNK = 80
NCHUNK = EPT // CHUNK
NPS = 632
NPS_LAST = N - 15 * NPS


def _sc_seg_pool(table, g2d, d2d, bf2d, D):
  mesh = plsc.VectorSubcoreMesh(core_axis_name="c", subcore_axis_name="s",
                                num_cores=2, num_subcores=16)
  nh = D // 16

  @functools.partial(
      pl.kernel,
      out_type=jax.ShapeDtypeStruct((2, N, D), jnp.float32),
      mesh=mesh,
      compiler_params=pltpu.CompilerParams(use_tc_tiling_on_sc=False),
      scratch_types=[
          pltpu.VMEM((NCHUNK, CHUNK), jnp.int32),
          pltpu.VMEM((NCHUNK, CHUNK), jnp.int32),
          pltpu.VMEM((NCHUNK, CHUNK), jnp.int32),
          pltpu.VMEM((CHUNK, D), jnp.float32),
          pltpu.VMEM((CHUNK, D), jnp.float32),
          pltpu.VMEM((CHUNK, D), jnp.float32),
          pltpu.VMEM((NPS, D), jnp.float32),
          pltpu.VMEM_SHARED((N, D), jnp.float32),
          pltpu.SemaphoreType.DMA,
          pltpu.SemaphoreType.DMA,
      ],
  )
  def k(tab_hbm, g_hbm, d_hbm, bf_hbm, out_hbm, gidx, didx, bfv, rows0,
        rows1, stage, zbuf, accum, sem0, sem1):
    cid = lax.axis_index("c")
    sid = lax.axis_index("s")
    t = cid * 16 + sid

    def zrow(i, c):
      for h in range(nh):
        zbuf[i, pl.ds(16 * h, 16)] = jnp.zeros((16,), jnp.float32)
      return c

    lax.fori_loop(0, NPS, zrow, 0)

    @pl.when(sid < 15)
    def _():
      pltpu.sync_copy(zbuf, accum.at[pl.ds(sid * NPS, NPS)])

    @pl.when(sid == 15)
    def _():
      pltpu.sync_copy(zbuf.at[pl.ds(0, NPS_LAST)],
                      accum.at[pl.ds(15 * NPS, NPS_LAST)])

    plsc.subcore_barrier()

    pltpu.sync_copy(g_hbm.at[t], gidx)
    pltpu.sync_copy(d_hbm.at[t], didx)
    pltpu.sync_copy(bf_hbm.at[t], bfv)

    pltpu.async_copy(tab_hbm.at[gidx.at[0]], rows0, sem0)

    def process(j, rows, sem_here, nxt_rows, sem_nxt, acc):
      @pl.when(j + 1 < NCHUNK)
      def _():
        pltpu.async_copy(tab_hbm.at[gidx.at[j + 1]], nxt_rows, sem_nxt)

      pltpu.make_async_copy(tab_hbm.at[gidx.at[j]], rows, sem_here).wait()
      z = jnp.zeros((16,), jnp.float32)
      for e16 in range(CHUNK // 16):
        bvec = bfv[j, pl.ds(16 * e16, 16)]
        for q in range(16):
          e = 16 * e16 + q
          b = bvec[q] != 0
          nacc = []
          for h in range(nh):
            a = acc[h] + rows[e, pl.ds(16 * h, 16)]
            stage[e, pl.ds(16 * h, 16)] = jnp.where(b, a, z)
            nacc.append(jnp.where(b, z, a))
          acc = tuple(nacc)
      pltpu.sync_copy(stage, accum.at[didx.at[j]], add=True)
      return acc

    acc = tuple(jnp.zeros((16,), jnp.float32) for _ in range(nh))

    def body2(i, acc):
      acc = process(2 * i, rows0, sem0, rows1, sem1, acc)
      acc = process(2 * i + 1, rows1, sem1, rows0, sem0, acc)
      return acc

    acc = lax.fori_loop(0, NCHUNK // 2, body2, acc)
    if NCHUNK % 2:
      process(NCHUNK - 1, rows0, sem0, rows1, sem1, acc)
    plsc.subcore_barrier()

    @pl.when(sid < 15)
    def _():
      sl = pl.ds(sid * NPS, NPS)
      pltpu.sync_copy(accum.at[sl], out_hbm.at[cid, sl])

    @pl.when(sid == 15)
    def _():
      sl = pl.ds(15 * NPS, NPS_LAST)
      pltpu.sync_copy(accum.at[sl], out_hbm.at[cid, sl])

  return k(table, g2d, d2d, bf2d)


def _tc_mlp(x, pools, W1f, b1, W2, b2, layer0):
  R = 2000
  Dx = x.shape[1]
  Dp = pools.shape[2]
  D1 = W1f.shape[0]

  def body(x_ref, p_ref, w1_ref, b1_ref, w2_ref, b2_ref, o_ref):
    nb = p_ref[0] + p_ref[1]
    xv = x_ref[...]
    if layer0:
      agg = jnp.concatenate([xv, nb], axis=1)
    else:
      agg = jnp.concatenate([xv, nb, nb + xv], axis=1)
    h = jnp.dot(agg.astype(jnp.bfloat16), w1_ref[...].astype(jnp.bfloat16),
                preferred_element_type=jnp.float32)
    h = jnp.maximum(h + b1_ref[...], 0.0)
    o_ref[...] = (
        jnp.dot(h.astype(jnp.bfloat16), w2_ref[...].astype(jnp.bfloat16),
                preferred_element_type=jnp.float32)
        + b2_ref[...])

  return pl.pallas_call(
      body,
      grid=(N // R,),
      in_specs=[
          pl.BlockSpec((R, Dx), lambda i: (i, 0)),
          pl.BlockSpec((2, R, Dp), lambda i: (0, i, 0)),
          pl.BlockSpec((D1, LATENT), lambda i: (0, 0)),
          pl.BlockSpec((1, LATENT), lambda i: (0, 0)),
          pl.BlockSpec((LATENT, LATENT), lambda i: (0, 0)),
          pl.BlockSpec((1, LATENT), lambda i: (0, 0)),
      ],
      out_specs=pl.BlockSpec((R, LATENT), lambda i: (i, 0)),
      out_shape=jax.ShapeDtypeStruct((N, LATENT), jnp.float32),
  )(x, pools, W1f, b1, W2, b2)


def _tc_final(wl2d, outg, WoutK, bout2d):
  def body(wl_ref, out_ref, w_ref, b_ref, o_ref):
    wl = wl_ref[...]
    vi = wl[:, None, :]
    vj = wl[:, :, None]
    ii = lax.broadcasted_iota(jnp.int32, (G, NPG, NPG), 2)
    jj = lax.broadcasted_iota(jnp.int32, (G, NPG, NPG), 1)
    cmp = (vj > vi) | ((vj == vi) & (jj < ii))
    rank = jnp.sum(cmp.astype(jnp.int32), axis=1)
    kk = lax.broadcasted_iota(jnp.int32, (G, K, NPG), 1)
    P = (rank[:, None, :] == kk).astype(jnp.bfloat16)
    sel = lax.dot_general(
        P, out_ref[...].astype(jnp.bfloat16),
        dimension_numbers=(((2,), (1,)), ((0,), (0,))),
        preferred_element_type=jnp.float32)
    acc = jnp.zeros((G, OUT_DIM), jnp.float32)
    for k in range(K):
      acc = acc + jnp.dot(sel[:, k, :].astype(jnp.bfloat16),
                          w_ref[k].astype(jnp.bfloat16),
                          preferred_element_type=jnp.float32)
    acc = jnp.maximum(acc + b_ref[...], 0.0)
    o_ref[...] = jnp.maximum(acc, 0.0)

  return pl.pallas_call(
      body,
      out_shape=jax.ShapeDtypeStruct((G, OUT_DIM), jnp.float32),
  )(wl2d, outg, WoutK, bout2d)


def kernel(node_feat, edge_feat, edge_index, W1_0, b1_0, W2_0, b2_0, W1, b1,
           W2, b2, Wout, bout):
  src = edge_index[0]
  dst = edge_index[1]

  order = jnp.argsort(dst, stable=True).astype(jnp.int32)
  ds = jnp.take(dst, order)
  ss = jnp.take(src, order)
  nxt = jnp.concatenate([ds[1:], jnp.full((1,), -1, jnp.int32)])
  bf = (ds != nxt).astype(jnp.int32)
  bf = bf.at[EPT - 1::EPT].set(1)

  shp = (NTILES, NCHUNK, CHUNK)
  p2d = order.reshape(shp)
  s2d = ss.reshape(shp)
  d2d = ds.reshape(shp)
  bf2d = bf.reshape(shp)

  e2n = _sc_seg_pool(edge_feat, p2d, d2d, bf2d, D_EDGE)
  ego = _tc_mlp(node_feat, e2n, W1_0, b1_0.reshape(1, -1), W2_0,
                b2_0.reshape(1, -1), layer0=True)
  cats = [ego]
  for l in range(NUM_LAYERS - 1):
    nbr = _sc_seg_pool(ego, s2d, d2d, bf2d, LATENT)
    ego = _tc_mlp(ego, nbr, W1[l], b1[l].reshape(1, -1), W2[l],
                  b2[l].reshape(1, -1), layer0=False)
    cats.append(ego)

  wl2d = ego[:, LATENT - 1].reshape(G, NPG)
  outg = jnp.concatenate(cats, axis=1).reshape(G, NPG, NUM_LAYERS * LATENT)
  WoutK = Wout.reshape(K, NUM_LAYERS * LATENT, OUT_DIM)
  return _tc_final(wl2d, outg, WoutK, bout.reshape(1, OUT_DIM))


# speedup vs baseline: 5.6000x; 1.0390x over previous
import functools

import jax
import jax.numpy as jnp
from jax import lax
from jax.experimental import pallas as pl
from jax.experimental.pallas import tpu as pltpu
from jax.experimental.pallas import tpu_sc as plsc

N = 10000
E = 320000
G = 100
NPG = 100
D_NODE = 128
D_EDGE = 16
LATENT = 32
NUM_LAYERS = 5
K = 30
OUT_DIM = 128

NTILES = 32
EPT = E // NTILES
CHUNK = 80
NCHUNK = EPT // CHUNK
NPS = 632
NPS_LAST = N - 15 * NPS


def _sc_seg_pool(table, g2d, d2d, bf2d, D):
  mesh = plsc.VectorSubcoreMesh(core_axis_name="c", subcore_axis_name="s",
                                num_cores=2, num_subcores=16)
  nh = D // 16

  @functools.partial(
      pl.kernel,
      out_type=jax.ShapeDtypeStruct((2, N, D), jnp.float32),
      mesh=mesh,
      compiler_params=pltpu.CompilerParams(use_tc_tiling_on_sc=False),
      scratch_types=[
          pltpu.VMEM((NCHUNK, CHUNK), jnp.int32),
          pltpu.VMEM((NCHUNK, CHUNK), jnp.int32),
          pltpu.VMEM((NCHUNK, CHUNK), jnp.int32),
          pltpu.VMEM((CHUNK, D), jnp.float32),
          pltpu.VMEM((CHUNK, D), jnp.float32),
          pltpu.VMEM((CHUNK, D), jnp.float32),
          pltpu.VMEM((CHUNK, D), jnp.float32),
          pltpu.VMEM((NPS, D), jnp.float32),
          pltpu.VMEM_SHARED((N, D), jnp.float32),
          pltpu.SemaphoreType.DMA,
          pltpu.SemaphoreType.DMA,
          pltpu.SemaphoreType.DMA,
          pltpu.SemaphoreType.DMA,
      ],
  )
  def k(tab_hbm, g_hbm, d_hbm, bf_hbm, out_hbm, gidx, didx, bfv, rows0,
        rows1, stage0, stage1, zbuf, accum, sem0, sem1, ssem0, ssem1):
    cid = lax.axis_index("c")
    sid = lax.axis_index("s")
    t = cid * 16 + sid

    def zrow(i, c):
      for h in range(nh):
        zbuf[i, pl.ds(16 * h, 16)] = jnp.zeros((16,), jnp.float32)
      return c

    lax.fori_loop(0, NPS, zrow, 0)

    @pl.when(sid < 15)
    def _():
      pltpu.sync_copy(zbuf, accum.at[pl.ds(sid * NPS, NPS)])

    @pl.when(sid == 15)
    def _():
      pltpu.sync_copy(zbuf.at[pl.ds(0, NPS_LAST)],
                      accum.at[pl.ds(15 * NPS, NPS_LAST)])

    plsc.subcore_barrier()

    pltpu.sync_copy(g_hbm.at[t], gidx)
    pltpu.sync_copy(d_hbm.at[t], didx)
    pltpu.sync_copy(bf_hbm.at[t], bfv)

    pltpu.async_copy(tab_hbm.at[gidx.at[0]], rows0, sem0)

    def process(j, rows, sem_here, nxt_rows, sem_nxt, stage, ssem, acc):
      @pl.when(j + 1 < NCHUNK)
      def _():
        pltpu.async_copy(tab_hbm.at[gidx.at[j + 1]], nxt_rows, sem_nxt)

      pltpu.make_async_copy(tab_hbm.at[gidx.at[j]], rows, sem_here).wait()

      @pl.when(j >= 2)
      def _():
        pltpu.make_async_copy(stage, accum.at[didx.at[j - 2]], ssem).wait()

      z = jnp.zeros((16,), jnp.float32)
      for e16 in range(CHUNK // 16):
        bvec = bfv[j, pl.ds(16 * e16, 16)]
        for q in range(16):
          e = 16 * e16 + q
          b = bvec[q] != 0
          nacc = []
          for h in range(nh):
            a = acc[h] + rows[e, pl.ds(16 * h, 16)]
            stage[e, pl.ds(16 * h, 16)] = jnp.where(b, a, z)
            nacc.append(jnp.where(b, z, a))
          acc = tuple(nacc)
      pltpu.async_copy(stage, accum.at[didx.at[j]], ssem, add=True)
      return acc

    acc = tuple(jnp.zeros((16,), jnp.float32) for _ in range(nh))

    def body2(i, acc):
      acc = process(2 * i, rows0, sem0, rows1, sem1, stage0, ssem0, acc)
      acc = process(2 * i + 1, rows1, sem1, rows0, sem0, stage1, ssem1, acc)
      return acc

    acc = lax.fori_loop(0, NCHUNK // 2, body2, acc)
    if NCHUNK % 2:
      process(NCHUNK - 1, rows0, sem0, rows1, sem1, stage0, ssem0, acc)
    pltpu.make_async_copy(stage0, accum.at[didx.at[NCHUNK - 1]], ssem0).wait()
    pltpu.make_async_copy(stage1, accum.at[didx.at[NCHUNK - 2]], ssem1).wait()
    plsc.subcore_barrier()

    @pl.when(sid < 15)
    def _():
      sl = pl.ds(sid * NPS, NPS)
      pltpu.sync_copy(accum.at[sl], out_hbm.at[cid, sl])

    @pl.when(sid == 15)
    def _():
      sl = pl.ds(15 * NPS, NPS_LAST)
      pltpu.sync_copy(accum.at[sl], out_hbm.at[cid, sl])

  return k(table, g2d, d2d, bf2d)


def _tc_mlp(x, pools, W1f, b1, W2, b2, layer0):
  R = 2000
  Dx = x.shape[1]
  Dp = pools.shape[2]
  D1 = W1f.shape[0]

  def body(x_ref, p_ref, w1_ref, b1_ref, w2_ref, b2_ref, o_ref):
    nb = p_ref[0] + p_ref[1]
    xv = x_ref[...]
    if layer0:
      agg = jnp.concatenate([xv, nb], axis=1)
    else:
      agg = jnp.concatenate([xv, nb, nb + xv], axis=1)
    h = jnp.dot(agg.astype(jnp.bfloat16), w1_ref[...].astype(jnp.bfloat16),
                preferred_element_type=jnp.float32)
    h = jnp.maximum(h + b1_ref[...], 0.0)
    o_ref[...] = (
        jnp.dot(h.astype(jnp.bfloat16), w2_ref[...].astype(jnp.bfloat16),
                preferred_element_type=jnp.float32)
        + b2_ref[...])

  return pl.pallas_call(
      body,
      grid=(N // R,),
      in_specs=[
          pl.BlockSpec((R, Dx), lambda i: (i, 0)),
          pl.BlockSpec((2, R, Dp), lambda i: (0, i, 0)),
          pl.BlockSpec((D1, LATENT), lambda i: (0, 0)),
          pl.BlockSpec((1, LATENT), lambda i: (0, 0)),
          pl.BlockSpec((LATENT, LATENT), lambda i: (0, 0)),
          pl.BlockSpec((1, LATENT), lambda i: (0, 0)),
      ],
      out_specs=pl.BlockSpec((R, LATENT), lambda i: (i, 0)),
      out_shape=jax.ShapeDtypeStruct((N, LATENT), jnp.float32),
  )(x, pools, W1f, b1, W2, b2)


def _tc_final(wl2d, outg, WoutK, bout2d):
  def body(wl_ref, out_ref, w_ref, b_ref, o_ref):
    wl = wl_ref[...]
    vi = wl[:, None, :]
    vj = wl[:, :, None]
    ii = lax.broadcasted_iota(jnp.int32, (G, NPG, NPG), 2)
    jj = lax.broadcasted_iota(jnp.int32, (G, NPG, NPG), 1)
    cmp = (vj > vi) | ((vj == vi) & (jj < ii))
    rank = jnp.sum(cmp.astype(jnp.int32), axis=1)
    kk = lax.broadcasted_iota(jnp.int32, (G, K, NPG), 1)
    P = (rank[:, None, :] == kk).astype(jnp.bfloat16)
    sel = lax.dot_general(
        P, out_ref[...].astype(jnp.bfloat16),
        dimension_numbers=(((2,), (1,)), ((0,), (0,))),
        preferred_element_type=jnp.float32)
    acc = jnp.zeros((G, OUT_DIM), jnp.float32)
    for k in range(K):
      acc = acc + jnp.dot(sel[:, k, :].astype(jnp.bfloat16),
                          w_ref[k].astype(jnp.bfloat16),
                          preferred_element_type=jnp.float32)
    acc = jnp.maximum(acc + b_ref[...], 0.0)
    o_ref[...] = jnp.maximum(acc, 0.0)

  return pl.pallas_call(
      body,
      out_shape=jax.ShapeDtypeStruct((G, OUT_DIM), jnp.float32),
  )(wl2d, outg, WoutK, bout2d)


def kernel(node_feat, edge_feat, edge_index, W1_0, b1_0, W2_0, b2_0, W1, b1,
           W2, b2, Wout, bout):
  src = edge_index[0]
  dst = edge_index[1]

  order = jnp.argsort(dst, stable=True).astype(jnp.int32)
  ds = jnp.take(dst, order)
  ss = jnp.take(src, order)
  nxt = jnp.concatenate([ds[1:], jnp.full((1,), -1, jnp.int32)])
  bf = (ds != nxt).astype(jnp.int32)
  bf = bf.at[EPT - 1::EPT].set(1)

  shp = (NTILES, NCHUNK, CHUNK)
  p2d = order.reshape(shp)
  s2d = ss.reshape(shp)
  d2d = ds.reshape(shp)
  bf2d = bf.reshape(shp)

  e2n = _sc_seg_pool(edge_feat, p2d, d2d, bf2d, D_EDGE)
  ego = _tc_mlp(node_feat, e2n, W1_0, b1_0.reshape(1, -1), W2_0,
                b2_0.reshape(1, -1), layer0=True)
  cats = [ego]
  for l in range(NUM_LAYERS - 1):
    nbr = _sc_seg_pool(ego, s2d, d2d, bf2d, LATENT)
    ego = _tc_mlp(ego, nbr, W1[l], b1[l].reshape(1, -1), W2[l],
                  b2[l].reshape(1, -1), layer0=False)
    cats.append(ego)

  wl2d = ego[:, LATENT - 1].reshape(G, NPG)
  outg = jnp.concatenate(cats, axis=1).reshape(G, NPG, NUM_LAYERS * LATENT)
  WoutK = Wout.reshape(K, NUM_LAYERS * LATENT, OUT_DIM)
  return _tc_final(wl2d, outg, WoutK, bout.reshape(1, OUT_DIM))
